# Initial kernel scaffold; baseline (speedup 1.0000x reference)
import jax
import jax.numpy as jnp
from jax.experimental import pallas as pl


def kernel(inputs, W1, b1, W2, b2, neighbour_list):
    raise NotImplementedError("write your pallas kernel here")


# speedup vs baseline: 18.3653x; 18.3653x over previous
import functools

import jax
import jax.numpy as jnp
from jax import lax
from jax.experimental import pallas as pl
from jax.experimental.pallas import tpu as pltpu
from jax.experimental.pallas import tpu_sc as plsc

_LATENT = 16
_NSTEPS = 2

_NC, _NS = 2, 16
_NW = _NC * _NS

_R = 400000
_R_PAD = 425984
_ROWS_PER_W = _R_PAD // _NW
_CHUNK = 1024
_NCHUNK = _ROWS_PER_W // _CHUNK
_IDX_W = 128
_IPC = _CHUNK // _IDX_W

_PK = 8


def _pk_body(y_ref, w_ref, p0_ref, p1_ref, p2_ref):
    y = y_ref[...]
    w = w_ref[...]
    p0_ref[...] = jnp.dot(y, w[:, 0:128], preferred_element_type=jnp.float32)
    p1_ref[...] = jnp.dot(y, w[:, 128:256], preferred_element_type=jnp.float32)
    p2_ref[...] = jnp.dot(y, w[:, 256:384], preferred_element_type=jnp.float32)


def _make_pk(rp, dp, blkp):
    grid = rp // blkp
    return pl.pallas_call(
        _pk_body,
        grid=(grid,),
        in_specs=[
            pl.BlockSpec((blkp, dp), lambda i: (i, 0)),
            pl.BlockSpec((dp, 3 * 128), lambda i: (0, 0)),
        ],
        out_specs=[pl.BlockSpec((blkp, 128), lambda i: (i, 0))] * 3,
        out_shape=[jax.ShapeDtypeStruct((rp, 128), jnp.float32)] * 3,
    )


def _gather_sum_body(p0, p1, p2, i0, i1, i2, out_hbm,
                     iv0, iv1, iv2, r0, r1, r2, ov, sem):
    wid = lax.axis_index("s") * _NC + lax.axis_index("c")

    def chunk_body(c, carry):
        base = wid * _ROWS_PER_W + c * _CHUNK
        irow = wid * (_ROWS_PER_W // _IDX_W) + c * _IPC
        pltpu.sync_copy(i0.at[pl.ds(irow, _IPC)], iv0)
        pltpu.sync_copy(i1.at[pl.ds(irow, _IPC)], iv1)
        pltpu.sync_copy(i2.at[pl.ds(irow, _IPC)], iv2)
        cps = []
        for j in range(_IPC):
            dst = pl.ds(j * _IDX_W, _IDX_W)
            cps.append(pltpu.async_copy(p0.at[iv0.at[j]], r0.at[dst], sem))
            cps.append(pltpu.async_copy(p1.at[iv1.at[j]], r1.at[dst], sem))
            cps.append(pltpu.async_copy(p2.at[iv2.at[j]], r2.at[dst], sem))
        for cp in cps:
            cp.wait()

        def row_body(i, carry2):
            ov[i, :] = r0[i, :] + r1[i, :] + r2[i, :]
            return carry2

        lax.fori_loop(0, _CHUNK, row_body, 0, unroll=8)
        pltpu.sync_copy(ov, out_hbm.at[pl.ds(base, _CHUNK)])
        return carry

    lax.fori_loop(0, _NCHUNK, chunk_body, 0)


@functools.cache
def _get_gather_sum():
    return pl.kernel(
        _gather_sum_body,
        out_type=jax.ShapeDtypeStruct((_R_PAD, _LATENT), jnp.float32),
        mesh=plsc.VectorSubcoreMesh(core_axis_name="c", subcore_axis_name="s",
                                    num_cores=_NC, num_subcores=_NS),
        scratch_types=[
            pltpu.VMEM((_IPC, _IDX_W), jnp.int32),
            pltpu.VMEM((_IPC, _IDX_W), jnp.int32),
            pltpu.VMEM((_IPC, _IDX_W), jnp.int32),
            pltpu.VMEM((_CHUNK, _LATENT), jnp.float32),
            pltpu.VMEM((_CHUNK, _LATENT), jnp.float32),
            pltpu.VMEM((_CHUNK, _LATENT), jnp.float32),
            pltpu.VMEM((_CHUNK, _LATENT), jnp.float32),
            pltpu.SemaphoreType.DMA,
        ],
        compiler_params=pltpu.CompilerParams(use_tc_tiling_on_sc=False),
    )


def _upd_body(y_ref, g_ref, b1_ref, w2_ref, b2_ref, o_ref):
    h = jnp.tanh(g_ref[...] + b1_ref[...])
    f = jnp.dot(h, w2_ref[...], preferred_element_type=jnp.float32)
    o_ref[...] = y_ref[...] + f + b2_ref[...]


def _make_upd(rp, gp_rows, dp, blkp):
    grid = rp // blkp
    return pl.pallas_call(
        _upd_body,
        grid=(grid,),
        in_specs=[
            pl.BlockSpec((blkp, dp), lambda i: (i, 0)),
            pl.BlockSpec((blkp, 128), lambda i: (i, 0)),
            pl.BlockSpec((1, 128), lambda i: (0, 0)),
            pl.BlockSpec((128, dp), lambda i: (0, 0)),
            pl.BlockSpec((1, dp), lambda i: (0, 0)),
        ],
        out_specs=pl.BlockSpec((blkp, dp), lambda i: (i, 0)),
        out_shape=jax.ShapeDtypeStruct((rp, dp), jnp.float32),
    )


def kernel(inputs, W1, b1, W2, b2, neighbour_list):
    b, n, d = inputs.shape
    r = b * n
    rp = r // _PK
    dp = _PK * d

    ypk = inputs.reshape(rp, dp)

    offs = (jnp.arange(b, dtype=jnp.int32) * n)[:, None]
    idx = (neighbour_list.T[:, None, :] + offs[None]).reshape(3, r)
    idx = jnp.pad(idx, ((0, 0), (0, _R_PAD - r)))
    idx = idx.reshape(3, _R_PAD // _IDX_W, _IDX_W)
    i0, i1, i2 = idx[0], idx[1], idx[2]

    eye = jnp.eye(_PK, dtype=jnp.float32)
    w1k = W1.reshape(3, d, _LATENT)
    bigw1 = jnp.concatenate([jnp.kron(eye, w1k[k]) for k in range(3)], axis=1)
    w2p = jnp.pad(W2, ((0, 0), (0, d - _LATENT)))
    bigw2 = jnp.kron(eye, w2p)
    b1big = jnp.tile(b1, _PK).reshape(1, _PK * _LATENT)
    b2big = jnp.tile(jnp.pad(b2, (0, d - _LATENT)), _PK).reshape(1, dp)

    pk = _make_pk(rp, dp, 2000)
    upd = _make_upd(rp, _R_PAD // _PK, dp, 2000)
    gather = _get_gather_sum()

    for _ in range(_NSTEPS):
        p0, p1, p2 = pk(ypk, bigw1)
        g = gather(p0.reshape(r, _LATENT), p1.reshape(r, _LATENT),
                   p2.reshape(r, _LATENT), i0, i1, i2)
        gp = g.reshape(_R_PAD // _PK, 128)
        ypk = upd(ypk, gp, b1big, bigw2, b2big)
    return ypk.reshape(b, n, d)


# speedup vs baseline: 29.4424x; 1.6032x over previous
import functools

import jax
import jax.numpy as jnp
from jax import lax
from jax.experimental import pallas as pl
from jax.experimental.pallas import tpu as pltpu
from jax.experimental.pallas import tpu_sc as plsc

_LATENT = 16
_NSTEPS = 2

_NC, _NS = 2, 16
_NW = _NC * _NS

_B, _N, _D = 4, 100000, 20
_BLK = 1024
_NBLK = 98
_NPB = _NBLK * _BLK
_PRB = 13312
_VPB = _PRB * 8
_R_PAD = _B * _VPB
_ROWS_PER_W = _R_PAD // _NW
_CHUNK = 1024
_NCHUNK = _ROWS_PER_W // _CHUNK
_IDX_W = 128
_IPC = _CHUNK // _IDX_W


def _tables_body(q_ref, w1_ref, t0_ref, t1_ref, t2_ref):
    outs = (t0_ref, t1_ref, t2_ref)
    for bb in range(_B):
        qb = q_ref[:, bb, :]
        for k in range(3):
            w1k = w1_ref[k * _D:(k + 1) * _D, :]
            m = lax.dot_general(qb, w1k, (((0,), (0,)), ((), ())),
                                preferred_element_type=jnp.float32)
            outs[k][bb, :, :] = jnp.concatenate(
                [m[s * 128:(s + 1) * 128, :] for s in range(8)], axis=1)


def _make_tables():
    return pl.pallas_call(
        _tables_body,
        grid=(_NBLK,),
        in_specs=[
            pl.BlockSpec((_D, _B, _BLK), lambda i: (0, 0, i)),
            pl.BlockSpec((3 * _D, _LATENT), lambda i: (0, 0)),
        ],
        out_specs=[pl.BlockSpec((_B, 128, 128), lambda i: (0, i, 0))] * 3,
        out_shape=[jax.ShapeDtypeStruct((_B, _PRB, 128), jnp.float32)] * 3,
    )


def _gather_sum_body(p0, p1, p2, i0, i1, i2, out_hbm,
                     iv0, iv1, iv2, r0, r1, r2, ov, sem):
    wid = lax.axis_index("s") * _NC + lax.axis_index("c")

    def chunk_body(c, carry):
        base = wid * _ROWS_PER_W + c * _CHUNK
        irow = wid * (_ROWS_PER_W // _IDX_W) + c * _IPC
        pltpu.sync_copy(i0.at[pl.ds(irow, _IPC)], iv0)
        pltpu.sync_copy(i1.at[pl.ds(irow, _IPC)], iv1)
        pltpu.sync_copy(i2.at[pl.ds(irow, _IPC)], iv2)
        cps = []
        for j in range(_IPC):
            dst = pl.ds(j * _IDX_W, _IDX_W)
            cps.append(pltpu.async_copy(p0.at[iv0.at[j]], r0.at[dst], sem))
            cps.append(pltpu.async_copy(p1.at[iv1.at[j]], r1.at[dst], sem))
            cps.append(pltpu.async_copy(p2.at[iv2.at[j]], r2.at[dst], sem))
        for cp in cps:
            cp.wait()

        def row_body(i, carry2):
            ov[i, :] = r0[i, :] + r1[i, :] + r2[i, :]
            return carry2

        lax.fori_loop(0, _CHUNK, row_body, 0, unroll=8)
        pltpu.sync_copy(ov, out_hbm.at[pl.ds(base, _CHUNK)])
        return carry

    lax.fori_loop(0, _NCHUNK, chunk_body, 0)


@functools.cache
def _get_gather_sum():
    return pl.kernel(
        _gather_sum_body,
        out_type=jax.ShapeDtypeStruct((_R_PAD, _LATENT), jnp.float32),
        mesh=plsc.VectorSubcoreMesh(core_axis_name="c", subcore_axis_name="s",
                                    num_cores=_NC, num_subcores=_NS),
        scratch_types=[
            pltpu.VMEM((_IPC, _IDX_W), jnp.int32),
            pltpu.VMEM((_IPC, _IDX_W), jnp.int32),
            pltpu.VMEM((_IPC, _IDX_W), jnp.int32),
            pltpu.VMEM((_CHUNK, _LATENT), jnp.float32),
            pltpu.VMEM((_CHUNK, _LATENT), jnp.float32),
            pltpu.VMEM((_CHUNK, _LATENT), jnp.float32),
            pltpu.VMEM((_CHUNK, _LATENT), jnp.float32),
            pltpu.SemaphoreType.DMA,
        ],
        compiler_params=pltpu.CompilerParams(use_tc_tiling_on_sc=False),
    )


def _update_body(q_ref, g_ref, b1_ref, w2_ref, b2_ref, o_ref):
    for bb in range(_B):
        gb = g_ref[bb, :, :]
        mg = jnp.concatenate(
            [gb[:, s * 16:(s + 1) * 16] for s in range(8)], axis=0)
        h = jnp.tanh(mg + b1_ref[...])
        f = jnp.dot(h, w2_ref[...],
                    preferred_element_type=jnp.float32) + b2_ref[...]
        o_ref[:, bb, :] = q_ref[:, bb, :] + jnp.transpose(f, (1, 0))


def _make_update():
    return pl.pallas_call(
        _update_body,
        grid=(_NBLK,),
        in_specs=[
            pl.BlockSpec((_D, _B, _BLK), lambda i: (0, 0, i)),
            pl.BlockSpec((_B, 128, 128), lambda i: (0, i, 0)),
            pl.BlockSpec((1, _LATENT), lambda i: (0, 0)),
            pl.BlockSpec((_LATENT, _D), lambda i: (0, 0)),
            pl.BlockSpec((1, _D), lambda i: (0, 0)),
        ],
        out_specs=pl.BlockSpec((_D, _B, _BLK), lambda i: (0, 0, i)),
        out_shape=jax.ShapeDtypeStruct((_D, _B, _N), jnp.float32),
    )


def kernel(inputs, W1, b1, W2, b2, neighbour_list):
    b, n, d = inputs.shape
    qt = jnp.transpose(inputs, (2, 0, 1))

    nbr = jnp.pad(neighbour_list, ((0, _NPB - n), (0, 0)))
    j = nbr.T
    q = j % _BLK
    vloc = (j // _BLK) * _BLK + (q % 128) * 8 + q // 128
    vloc = vloc.reshape(3, _NBLK, 8, 128).swapaxes(2, 3).reshape(3, _NPB)
    vloc = jnp.pad(vloc, ((0, 0), (0, _VPB - _NPB)))
    boffs = (jnp.arange(_B, dtype=jnp.int32) * _VPB)[None, :, None]
    idx = (vloc[:, None, :] + boffs).reshape(3, _R_PAD // _IDX_W, _IDX_W)
    i0, i1, i2 = idx[0], idx[1], idx[2]

    w2p = jnp.pad(W2, ((0, 0), (0, d - _LATENT)))
    b1r = b1.reshape(1, _LATENT)
    b2r = jnp.pad(b2, (0, d - _LATENT)).reshape(1, d)

    tables = _make_tables()
    upd = _make_update()
    gather = _get_gather_sum()

    for _ in range(_NSTEPS):
        t0, t1, t2 = tables(qt, W1)
        g = gather(t0.reshape(_R_PAD, _LATENT), t1.reshape(_R_PAD, _LATENT),
                   t2.reshape(_R_PAD, _LATENT), i0, i1, i2)
        qt = upd(qt, g.reshape(_B, _PRB, 128), b1r, w2p, b2r)
    return jnp.transpose(qt, (1, 2, 0))


# speedup vs baseline: 37.7824x; 1.2833x over previous
import functools

import jax
import jax.numpy as jnp
from jax import lax
from jax.experimental import pallas as pl
from jax.experimental.pallas import tpu as pltpu
from jax.experimental.pallas import tpu_sc as plsc

_LATENT = 16
_NSTEPS = 2

_NC, _NS = 2, 16
_NW = _NC * _NS

_B, _N, _D = 4, 100000, 20
_BLK = 1024
_NBLK = 98
_NPB = _NBLK * _BLK
_PRB = 13312
_VPB = _PRB * 8
_R_PAD = _B * _VPB
_ROWS_PER_W = _R_PAD // _NW
_CHUNK = 1024
_NCHUNK = _ROWS_PER_W // _CHUNK
_IDX_W = 128
_IPC = _CHUNK // _IDX_W


def _tables_body(q_ref, w1_ref, t0_ref, t1_ref, t2_ref):
    outs = (t0_ref, t1_ref, t2_ref)
    for bb in range(_B):
        qb = q_ref[:, bb, :]
        for k in range(3):
            w1k = w1_ref[k * _D:(k + 1) * _D, :]
            m = lax.dot_general(qb, w1k, (((0,), (0,)), ((), ())),
                                preferred_element_type=jnp.float32)
            outs[k][bb, :, :] = jnp.concatenate(
                [m[s * 128:(s + 1) * 128, :] for s in range(8)], axis=1)


def _make_tables():
    return pl.pallas_call(
        _tables_body,
        grid=(_NBLK,),
        in_specs=[
            pl.BlockSpec((_D, _B, _BLK), lambda i: (0, 0, i)),
            pl.BlockSpec((3 * _D, _LATENT), lambda i: (0, 0)),
        ],
        out_specs=[pl.BlockSpec((_B, 128, 128), lambda i: (0, i, 0))] * 3,
        out_shape=[jax.ShapeDtypeStruct((_B, _PRB, 128), jnp.float32)] * 3,
    )


def _gather_sum_body(p0, p1, p2, i0, i1, i2, out_hbm,
                     iv, rv, ov, sem):
    wid = lax.axis_index("s") * _NC + lax.axis_index("c")
    tabs = (p0, p1, p2)
    idxs = (i0, i1, i2)

    def issue(c, buf):
        irow = wid * (_ROWS_PER_W // _IDX_W) + c * _IPC
        for k in range(3):
            pltpu.sync_copy(idxs[k].at[pl.ds(irow, _IPC)], iv.at[buf, k])
        for j in range(_IPC):
            for k in range(3):
                pltpu.async_copy(tabs[k].at[iv.at[buf, k, j]],
                                 rv.at[buf, k, pl.ds(j * _IDX_W, _IDX_W)],
                                 sem)

    def drain(buf):
        for j in range(_IPC):
            for k in range(3):
                pltpu.make_async_copy(
                    tabs[k].at[iv.at[buf, k, j]],
                    rv.at[buf, k, pl.ds(j * _IDX_W, _IDX_W)], sem).wait()

    def sum_store(c, buf):
        def row_body(i, carry2):
            ov[i, :] = (rv[buf, 0, i, :] + rv[buf, 1, i, :]
                        + rv[buf, 2, i, :])
            return carry2

        lax.fori_loop(0, _CHUNK, row_body, 0, unroll=8)
        base = wid * _ROWS_PER_W + c * _CHUNK
        pltpu.sync_copy(ov, out_hbm.at[pl.ds(base, _CHUNK)])

    issue(0, 0)

    def outer(t, carry):
        c2 = 2 * t
        drain(0)
        issue(c2 + 1, 1)
        sum_store(c2, 0)
        drain(1)
        issue(c2 + 2, 0)
        sum_store(c2 + 1, 1)
        return carry

    lax.fori_loop(0, (_NCHUNK - 1) // 2, outer, 0)
    drain(0)
    sum_store(_NCHUNK - 1, 0)


@functools.cache
def _get_gather_sum():
    return pl.kernel(
        _gather_sum_body,
        out_type=jax.ShapeDtypeStruct((_R_PAD, _LATENT), jnp.float32),
        mesh=plsc.VectorSubcoreMesh(core_axis_name="c", subcore_axis_name="s",
                                    num_cores=_NC, num_subcores=_NS),
        scratch_types=[
            pltpu.VMEM((2, 3, _IPC, _IDX_W), jnp.int32),
            pltpu.VMEM((2, 3, _CHUNK, _LATENT), jnp.float32),
            pltpu.VMEM((_CHUNK, _LATENT), jnp.float32),
            pltpu.SemaphoreType.DMA,
        ],
        compiler_params=pltpu.CompilerParams(use_tc_tiling_on_sc=False),
    )


def _update_body(q_ref, g_ref, b1_ref, w2_ref, b2_ref, o_ref):
    for bb in range(_B):
        gb = g_ref[bb, :, :]
        mg = jnp.concatenate(
            [gb[:, s * 16:(s + 1) * 16] for s in range(8)], axis=0)
        h = jnp.tanh(mg + b1_ref[...])
        f = jnp.dot(h, w2_ref[...],
                    preferred_element_type=jnp.float32) + b2_ref[...]
        o_ref[:, bb, :] = q_ref[:, bb, :] + jnp.transpose(f, (1, 0))


def _make_update():
    return pl.pallas_call(
        _update_body,
        grid=(_NBLK,),
        in_specs=[
            pl.BlockSpec((_D, _B, _BLK), lambda i: (0, 0, i)),
            pl.BlockSpec((_B, 128, 128), lambda i: (0, i, 0)),
            pl.BlockSpec((1, _LATENT), lambda i: (0, 0)),
            pl.BlockSpec((_LATENT, _D), lambda i: (0, 0)),
            pl.BlockSpec((1, _D), lambda i: (0, 0)),
        ],
        out_specs=pl.BlockSpec((_D, _B, _BLK), lambda i: (0, 0, i)),
        out_shape=jax.ShapeDtypeStruct((_D, _B, _N), jnp.float32),
    )


def kernel(inputs, W1, b1, W2, b2, neighbour_list):
    b, n, d = inputs.shape
    qt = jnp.transpose(inputs, (2, 0, 1))

    padrows = jnp.broadcast_to(
        (jnp.arange(_NPB - n, dtype=jnp.int32) * 997 % n)[:, None],
        (_NPB - n, 3))
    nbr = jnp.concatenate([neighbour_list, padrows], axis=0)
    j = nbr.T
    q = j % _BLK
    vloc = (j // _BLK) * _BLK + (q % 128) * 8 + q // 128
    vloc = vloc.reshape(3, _NBLK, 8, 128).swapaxes(2, 3).reshape(3, _NPB)
    padv = jnp.broadcast_to(
        (jnp.arange(_VPB - _NPB, dtype=jnp.int32) * 1013 % _NPB)[None, :],
        (3, _VPB - _NPB))
    vloc = jnp.concatenate([vloc, padv], axis=1)
    boffs = (jnp.arange(_B, dtype=jnp.int32) * _VPB)[None, :, None]
    idx = (vloc[:, None, :] + boffs).reshape(3, _R_PAD // _IDX_W, _IDX_W)
    i0, i1, i2 = idx[0], idx[1], idx[2]

    w2p = jnp.pad(W2, ((0, 0), (0, d - _LATENT)))
    b1r = b1.reshape(1, _LATENT)
    b2r = jnp.pad(b2, (0, d - _LATENT)).reshape(1, d)

    tables = _make_tables()
    upd = _make_update()
    gather = _get_gather_sum()

    for _ in range(_NSTEPS):
        t0, t1, t2 = tables(qt, W1)
        g = gather(t0.reshape(_R_PAD, _LATENT), t1.reshape(_R_PAD, _LATENT),
                   t2.reshape(_R_PAD, _LATENT), i0, i1, i2)
        qt = upd(qt, g.reshape(_B, _PRB, 128), b1r, w2p, b2r)
    return jnp.transpose(qt, (1, 2, 0))
